# Initial kernel scaffold; baseline (speedup 1.0000x reference)
import jax
import jax.numpy as jnp
from jax.experimental import pallas as pl


def kernel(h_node, edge_index, h_edge, W1, b1, g1, be1, W2, b2, g2, be2, W_ih, W_hh, b_ih, b_hh):
    raise NotImplementedError("write your pallas kernel here")


# speedup vs baseline: 1.7290x; 1.7290x over previous
import functools

import jax
import jax.numpy as jnp
from jax import lax
from jax.experimental import pallas as pl
from jax.experimental.pallas import tpu as pltpu
from jax.experimental.pallas import tpu_sc as plsc

_N = 10000
_E = 160000
_D = 32
_DE = 16
_N_LAYERS = 2
_EPS = 1e-5
_SLOPE = 0.8

_NC = 2
_NS = 16
_NW = _NC * _NS
_C = 128
_NCH = 40
_EW = _NCH * _C
_EP = _NW * _EW
_NP = _N + 16
_NPS = _N // _NS

_TE = 2000
_TEP = 2048
_TN = 2000


def _moments_body(x_ref, gram_ref, colsum_ref):
    @pl.when(pl.program_id(0) == 0)
    def _():
        gram_ref[...] = jnp.zeros_like(gram_ref)
        colsum_ref[...] = jnp.zeros_like(colsum_ref)

    x = x_ref[...]
    gram_ref[...] += lax.dot_general(
        x, x, (((0,), (0,)), ((), ())), preferred_element_type=jnp.float32)
    colsum_ref[...] += jnp.sum(x, axis=0, keepdims=True)


def _moments(x, d):
    grid = (x.shape[0] // _TE,)
    return pl.pallas_call(
        _moments_body,
        grid=grid,
        in_specs=[pl.BlockSpec((_TE, d), lambda i: (i, 0))],
        out_specs=[pl.BlockSpec((d, d), lambda i: (0, 0)),
                   pl.BlockSpec((1, d), lambda i: (0, 0))],
        out_shape=[jax.ShapeDtypeStruct((d, d), jnp.float32),
                   jax.ShapeDtypeStruct((1, d), jnp.float32)],
    )(x)


def _x1_body(he_ref, w_ref, b_ref, x1_ref, gram_ref, colsum_ref):
    @pl.when(pl.program_id(0) == 0)
    def _():
        gram_ref[...] = jnp.zeros_like(gram_ref)
        colsum_ref[...] = jnp.zeros_like(colsum_ref)

    y = jnp.dot(he_ref[...], w_ref[...],
                preferred_element_type=jnp.float32) + b_ref[...]
    x1 = jnp.where(y >= 0, y, _SLOPE * y)
    x1_ref[...] = x1
    gram_ref[...] += lax.dot_general(
        x1, x1, (((0,), (0,)), ((), ())), preferred_element_type=jnp.float32)
    colsum_ref[...] += jnp.sum(x1, axis=0, keepdims=True)


def _x1_compute(h_edge, w1f, b1f):
    grid = (_E // _TE,)
    return pl.pallas_call(
        _x1_body,
        grid=grid,
        in_specs=[pl.BlockSpec((_TE, _DE), lambda i: (i, 0)),
                  pl.BlockSpec((_DE, _DE), lambda i: (0, 0)),
                  pl.BlockSpec((1, _DE), lambda i: (0, 0))],
        out_specs=[pl.BlockSpec((_TE, _DE), lambda i: (i, 0)),
                   pl.BlockSpec((_DE, _DE), lambda i: (0, 0)),
                   pl.BlockSpec((1, _DE), lambda i: (0, 0))],
        out_shape=[jax.ShapeDtypeStruct((_E, _DE), jnp.float32),
                   jax.ShapeDtypeStruct((_DE, _DE), jnp.float32),
                   jax.ShapeDtypeStruct((1, _DE), jnp.float32)],
    )(h_edge, w1f, b1f)


def _edge_body(hs_ref, x1_ref, ab_ref, cm_ref, m_ref):
    hs = hs_ref[...]
    x1 = x1_ref[...]
    z = jnp.dot(hs, ab_ref[...],
                preferred_element_type=jnp.float32)
    acc = jnp.dot(hs, cm_ref[...],
                  preferred_element_type=jnp.float32)
    for k in range(_DE):
        acc += x1[:, k:k + 1] * z[:, k * _D:(k + 1) * _D]
    m_ref[...] = acc


def _edge_messages(hsrc, x1, abig, cmat):
    grid = (_EP // _TEP,)
    return pl.pallas_call(
        _edge_body,
        grid=grid,
        in_specs=[pl.BlockSpec((_TEP, _D), lambda i: (i, 0)),
                  pl.BlockSpec((_TEP, _DE), lambda i: (i, 0)),
                  pl.BlockSpec((_D, _DE * _D), lambda i: (0, 0)),
                  pl.BlockSpec((_D, _D), lambda i: (0, 0))],
        out_specs=pl.BlockSpec((_TEP, _D), lambda i: (i, 0)),
        out_shape=jax.ShapeDtypeStruct((_EP, _D), jnp.float32),
    )(hsrc, x1, abig, cmat)


def _gru_body(s0_ref, s1_ref, c0_ref, c1_ref, h_ref,
              wih_ref, whh_ref, bih_ref, bhh_ref, out_ref):
    cnt = jnp.maximum(c0_ref[...] + c1_ref[...], 1.0)
    mag = (s0_ref[...] + s1_ref[...]) / cnt
    h = h_ref[...]
    gi = jnp.dot(mag, wih_ref[...],
                 preferred_element_type=jnp.float32) + bih_ref[...]
    gh = jnp.dot(h, whh_ref[...],
                 preferred_element_type=jnp.float32) + bhh_ref[...]
    r = jax.nn.sigmoid(gi[:, 0:_D] + gh[:, 0:_D])
    zg = jax.nn.sigmoid(gi[:, _D:2 * _D] + gh[:, _D:2 * _D])
    n = jnp.tanh(gi[:, 2 * _D:3 * _D] + r * gh[:, 2 * _D:3 * _D])
    out_ref[...] = (1.0 - zg) * n + zg * h


def _gru(s0, s1, c0, c1, h, wiht, whht, bih2, bhh2):
    grid = (_N // _TN,)
    blk = lambda i: (i, 0)
    full = lambda i: (0, 0)
    return pl.pallas_call(
        _gru_body,
        grid=grid,
        in_specs=[pl.BlockSpec((_TN, _D), blk),
                  pl.BlockSpec((_TN, _D), blk),
                  pl.BlockSpec((_TN, _D), blk),
                  pl.BlockSpec((_TN, _D), blk),
                  pl.BlockSpec((_TN, _D), blk),
                  pl.BlockSpec((_D, 3 * _D), full),
                  pl.BlockSpec((_D, 3 * _D), full),
                  pl.BlockSpec((1, 3 * _D), full),
                  pl.BlockSpec((1, 3 * _D), full)],
        out_specs=pl.BlockSpec((_TN, _D), blk),
        out_shape=jax.ShapeDtypeStruct((_N, _D), jnp.float32),
    )(s0, s1, c0, c1, h, wiht, whht, bih2, bhh2)


def _sc_mesh():
    return plsc.VectorSubcoreMesh(core_axis_name="c", subcore_axis_name="s")


_SC_PARAMS = pltpu.CompilerParams(use_tc_tiling_on_sc=False)


def _gather_rows(h, src3):
    @functools.partial(
        pl.kernel,
        mesh=_sc_mesh(),
        out_type=jax.ShapeDtypeStruct((_EP, _D), jnp.float32),
        scratch_types=[
            pltpu.VMEM((_NCH, _C), jnp.int32),
            pltpu.VMEM((_C, _D), jnp.float32),
            pltpu.SemaphoreType.DMA,
        ],
        compiler_params=_SC_PARAMS,
    )
    def k(h_hbm, src_hbm, out_hbm, idx_v, rows_v, sem):
        c = lax.axis_index("c")
        s = lax.axis_index("s")
        wid = s * _NC + c
        pltpu.sync_copy(src_hbm.at[wid], idx_v)
        base = wid * _EW

        def body(j, carry):
            pltpu.async_copy(h_hbm.at[idx_v.at[j]], rows_v, sem).wait()
            pltpu.sync_copy(rows_v, out_hbm.at[pl.ds(base + j * _C, _C)])
            return carry

        lax.fori_loop(0, _NCH, body, 0)

    return k(h, src3)


def _scatter_rows(rows, dst3, zeros_nd):
    def body(m_hbm, dst_hbm, zero_hbm, out_hbm, idx_v, rows_v, acc):
        c = lax.axis_index("c")
        s = lax.axis_index("s")
        wid = s * _NC + c
        pltpu.sync_copy(zero_hbm.at[pl.ds(s * _NPS, _NPS)],
                        acc.at[pl.ds(s * _NPS, _NPS)])
        pltpu.sync_copy(dst_hbm.at[wid], idx_v)
        plsc.subcore_barrier()
        base = wid * _EW

        def loop(j, carry):
            pltpu.sync_copy(m_hbm.at[pl.ds(base + j * _C, _C)], rows_v)
            pltpu.sync_copy(rows_v, acc.at[idx_v.at[j]], add=True)
            return carry

        lax.fori_loop(0, _NCH, loop, 0)
        plsc.subcore_barrier()
        pltpu.sync_copy(acc.at[pl.ds(s * _NPS, _NPS)],
                        out_hbm.at[c, pl.ds(s * _NPS, _NPS)])

    k = functools.partial(
        pl.kernel,
        mesh=_sc_mesh(),
        out_type=jax.ShapeDtypeStruct((_NC, _N, _D), jnp.float32),
        scratch_types=[
            pltpu.VMEM((_NCH, _C), jnp.int32),
            pltpu.VMEM((_C, _D), jnp.float32),
            pltpu.VMEM_SHARED((_N, _D), jnp.float32),
        ],
        compiler_params=_SC_PARAMS,
    )(body)
    return k(rows, dst3, zeros_nd)


def kernel(h_node, edge_index, h_edge, W1, b1, g1, be1, W2, b2, g2, be2,
           W_ih, W_hh, b_ih, b_hh):
    f32 = jnp.float32
    h_node = h_node.astype(f32)
    h_edge = h_edge.astype(f32)
    pad_e = _EP - _E
    src3 = jnp.concatenate(
        [edge_index[0], jnp.full((pad_e,), _N, jnp.int32)]).reshape(
            _NW, _NCH, _C)
    dst3 = jnp.concatenate(
        [edge_index[1], jnp.zeros((pad_e,), jnp.int32)]).reshape(
            _NW, _NCH, _C)

    ecount = jnp.float32(_E)

    gram_h, colsum_h = _moments(h_edge, _DE)
    mh = (colsum_h[0] / ecount)
    sh = gram_h / ecount
    hp_ = jax.lax.Precision.HIGHEST
    w1mh = jnp.einsum('ck,k->c', W1, mh, precision=hp_)
    mu1 = w1mh + b1
    ey1sq = jnp.einsum('ck,kl,cl->c', W1, sh, W1, precision=hp_) \
        + 2.0 * b1 * w1mh + b1 * b1
    var1 = ey1sq - mu1 * mu1
    s1v = g1 / jnp.sqrt(var1 + _EPS)
    w1f = W1.T * s1v[None, :]
    b1f = ((b1 - mu1) * s1v + be1)[None, :]

    x1, gram_x, colsum_x = _x1_compute(h_edge, w1f, b1f)
    mx = colsum_x[0] / ecount
    sx = gram_x / ecount
    w2mx = jnp.einsum('ck,k->c', W2, mx, precision=hp_)
    mu2 = w2mx + b2
    ey2sq = jnp.einsum('ck,kl,cl->c', W2, sx, W2, precision=hp_) \
        + 2.0 * b2 * w2mx + b2 * b2
    var2 = ey2sq - mu2 * mu2
    alpha = g2 / jnp.sqrt(var2 + _EPS)
    cc = alpha * b2 + be2 - alpha * mu2
    a2 = (alpha[:, None] * W2).reshape(_D, _D, _DE)
    abig = jnp.transpose(a2, (0, 2, 1)).reshape(_D, _DE * _D)
    cmat = cc.reshape(_D, _D)

    zeros_nd = jnp.zeros((_N, _D), f32)
    x1p = jnp.concatenate([x1, jnp.zeros((pad_e, _DE), f32)])

    ones_ep = jnp.concatenate(
        [jnp.ones((_E, _D), f32), jnp.zeros((pad_e, _D), f32)])
    cpart = _scatter_rows(ones_ep, dst3, zeros_nd)
    c0, c1 = cpart[0], cpart[1]

    wiht = W_ih.T.astype(f32)
    whht = W_hh.T.astype(f32)
    bih2 = b_ih[None, :].astype(f32)
    bhh2 = b_hh[None, :].astype(f32)

    h = h_node
    for _ in range(_N_LAYERS):
        hp = jnp.concatenate([h, jnp.zeros((_NP - _N, _D), f32)])
        hsrc = _gather_rows(hp, src3)
        m = _edge_messages(hsrc, x1p, abig, cmat)
        spart = _scatter_rows(m, dst3, zeros_nd)
        h = _gru(spart[0], spart[1], c0, c1, h, wiht, whht, bih2, bhh2)
    return h


# speedup vs baseline: 3.7581x; 2.1736x over previous
import functools

import jax
import jax.numpy as jnp
from jax import lax
from jax.experimental import pallas as pl
from jax.experimental.pallas import tpu as pltpu
from jax.experimental.pallas import tpu_sc as plsc

_N = 10000
_E = 160000
_D = 32
_DE = 16
_N_LAYERS = 2
_EPS = 1e-5
_SLOPE = 0.8

_NC = 2
_NS = 16
_NW = _NC * _NS
_C = 128
_NCH = 40
_EW = _NCH * _C
_EP = _NW * _EW
_NCH_LAST = (_E - (_NW - 1) * _EW) // _C
_NPS = _N // _NS

_TE = 2000
_TEP = 4096
_TN = 2000


def _moments_body(x_ref, gram_ref, colsum_ref):
    @pl.when(pl.program_id(0) == 0)
    def _():
        gram_ref[...] = jnp.zeros_like(gram_ref)
        colsum_ref[...] = jnp.zeros_like(colsum_ref)

    x = x_ref[...]
    gram_ref[...] += lax.dot_general(
        x, x, (((0,), (0,)), ((), ())), preferred_element_type=jnp.float32)
    colsum_ref[...] += jnp.sum(x, axis=0, keepdims=True)


def _moments(x, d):
    grid = (x.shape[0] // _TE,)
    return pl.pallas_call(
        _moments_body,
        grid=grid,
        in_specs=[pl.BlockSpec((_TE, d), lambda i: (i, 0))],
        out_specs=[pl.BlockSpec((d, d), lambda i: (0, 0)),
                   pl.BlockSpec((1, d), lambda i: (0, 0))],
        out_shape=[jax.ShapeDtypeStruct((d, d), jnp.float32),
                   jax.ShapeDtypeStruct((1, d), jnp.float32)],
    )(x)


def _x1_body(he_ref, w_ref, b_ref, x1_ref, gram_ref, colsum_ref):
    @pl.when(pl.program_id(0) == 0)
    def _():
        gram_ref[...] = jnp.zeros_like(gram_ref)
        colsum_ref[...] = jnp.zeros_like(colsum_ref)

    y = jnp.dot(he_ref[...], w_ref[...],
                preferred_element_type=jnp.float32) + b_ref[...]
    x1 = jnp.where(y >= 0, y, _SLOPE * y)
    x1_ref[...] = x1
    gram_ref[...] += lax.dot_general(
        x1, x1, (((0,), (0,)), ((), ())), preferred_element_type=jnp.float32)
    colsum_ref[...] += jnp.sum(x1, axis=0, keepdims=True)


def _x1_compute(h_edge, w1f, b1f):
    grid = (_E // _TE,)
    return pl.pallas_call(
        _x1_body,
        grid=grid,
        in_specs=[pl.BlockSpec((_TE, _DE), lambda i: (i, 0)),
                  pl.BlockSpec((_DE, _DE), lambda i: (0, 0)),
                  pl.BlockSpec((1, _DE), lambda i: (0, 0))],
        out_specs=[pl.BlockSpec((_TE, _DE), lambda i: (i, 0)),
                   pl.BlockSpec((_DE, _DE), lambda i: (0, 0)),
                   pl.BlockSpec((1, _DE), lambda i: (0, 0))],
        out_shape=[jax.ShapeDtypeStruct((_EP, _DE), jnp.float32),
                   jax.ShapeDtypeStruct((_DE, _DE), jnp.float32),
                   jax.ShapeDtypeStruct((1, _DE), jnp.float32)],
    )(h_edge, w1f, b1f)


def _edge_body(hs_ref, x1_ref, rx_ref, rh_ref, b_ref, cm_ref, m_ref):
    hs = hs_ref[...]
    x1 = x1_ref[...]
    xr = jnp.dot(x1, rx_ref[...],
                 preferred_element_type=jnp.float32)
    hr = jnp.dot(hs, rh_ref[...],
                 preferred_element_type=jnp.float32)
    p = xr * hr
    m_ref[...] = (
        jnp.dot(p, b_ref[...], preferred_element_type=jnp.float32)
        + jnp.dot(hs, cm_ref[...], preferred_element_type=jnp.float32))


def _edge_messages(hsrc, x1, rx, rh, bmat, cmat):
    grid = (_EP // _TEP,)
    return pl.pallas_call(
        _edge_body,
        grid=grid,
        in_specs=[pl.BlockSpec((_TEP, _D), lambda i: (i, 0)),
                  pl.BlockSpec((_TEP, _DE), lambda i: (i, 0)),
                  pl.BlockSpec((_DE, _DE * _D), lambda i: (0, 0)),
                  pl.BlockSpec((_D, _DE * _D), lambda i: (0, 0)),
                  pl.BlockSpec((_DE * _D, _D), lambda i: (0, 0)),
                  pl.BlockSpec((_D, _D), lambda i: (0, 0))],
        out_specs=pl.BlockSpec((_TEP, _D), lambda i: (i, 0)),
        out_shape=jax.ShapeDtypeStruct((_EP, _D), jnp.float32),
    )(hsrc, x1, rx, rh, bmat, cmat)


def _gru_body(s0_ref, s1_ref, c0_ref, c1_ref, h_ref,
              wih_ref, whh_ref, bih_ref, bhh_ref, out_ref):
    cnt = jnp.maximum(c0_ref[...] + c1_ref[...], 1.0)
    mag = (s0_ref[...] + s1_ref[...]) / cnt
    h = h_ref[...]
    gi = jnp.dot(mag, wih_ref[...],
                 preferred_element_type=jnp.float32) + bih_ref[...]
    gh = jnp.dot(h, whh_ref[...],
                 preferred_element_type=jnp.float32) + bhh_ref[...]
    r = jax.nn.sigmoid(gi[:, 0:_D] + gh[:, 0:_D])
    zg = jax.nn.sigmoid(gi[:, _D:2 * _D] + gh[:, _D:2 * _D])
    n = jnp.tanh(gi[:, 2 * _D:3 * _D] + r * gh[:, 2 * _D:3 * _D])
    out_ref[...] = (1.0 - zg) * n + zg * h


def _gru(s0, s1, c0, c1, h, wiht, whht, bih2, bhh2):
    grid = (_N // _TN,)
    blk = lambda i: (i, 0)
    full = lambda i: (0, 0)
    return pl.pallas_call(
        _gru_body,
        grid=grid,
        in_specs=[pl.BlockSpec((_TN, _D), blk),
                  pl.BlockSpec((_TN, _D), blk),
                  pl.BlockSpec((_TN, _D), blk),
                  pl.BlockSpec((_TN, _D), blk),
                  pl.BlockSpec((_TN, _D), blk),
                  pl.BlockSpec((_D, 3 * _D), full),
                  pl.BlockSpec((_D, 3 * _D), full),
                  pl.BlockSpec((1, 3 * _D), full),
                  pl.BlockSpec((1, 3 * _D), full)],
        out_specs=pl.BlockSpec((_TN, _D), blk),
        out_shape=jax.ShapeDtypeStruct((_N, _D), jnp.float32),
    )(s0, s1, c0, c1, h, wiht, whht, bih2, bhh2)


def _sc_mesh():
    return plsc.VectorSubcoreMesh(core_axis_name="c", subcore_axis_name="s")


_SC_PARAMS = pltpu.CompilerParams(use_tc_tiling_on_sc=False)


def _num_chunks(wid):
    return jnp.where(wid == _NW - 1, _NCH_LAST, _NCH)


def _gather_rows(h, src2):
    @functools.partial(
        pl.kernel,
        mesh=_sc_mesh(),
        out_type=jax.ShapeDtypeStruct((_EP, _D), jnp.float32),
        scratch_types=[
            pltpu.VMEM((_NCH, _C), jnp.int32),
            pltpu.VMEM((_C, _D), jnp.float32),
            pltpu.SemaphoreType.DMA,
        ],
        compiler_params=_SC_PARAMS,
    )
    def k(h_hbm, src_hbm, out_hbm, idx_v, rows_v, sem):
        c = lax.axis_index("c")
        s = lax.axis_index("s")
        wid = s * _NC + c
        pltpu.sync_copy(src_hbm.at[pl.ds(wid * _NCH, _NCH)], idx_v)
        base = wid * _EW

        def body(j, carry):
            pltpu.async_copy(h_hbm.at[idx_v.at[j]], rows_v, sem).wait()
            pltpu.sync_copy(rows_v, out_hbm.at[pl.ds(base + j * _C, _C)])
            return carry

        lax.fori_loop(0, _num_chunks(wid), body, 0)

    return k(h, src2)


def _scatter_rows(rows, dst2, zeros_nd, ones_rows):
    counts_mode = rows is None
    operands = (dst2, zeros_nd, ones_rows) if counts_mode else (
        rows, dst2, zeros_nd)

    def body(*refs):
        if counts_mode:
            dst_hbm, zero_hbm, ones_hbm, out_hbm, idx_v, rows_v, acc = refs
        else:
            m_hbm, dst_hbm, zero_hbm, out_hbm, idx_v, rows_v, acc = refs
        c = lax.axis_index("c")
        s = lax.axis_index("s")
        wid = s * _NC + c
        pltpu.sync_copy(zero_hbm.at[pl.ds(s * _NPS, _NPS)],
                        acc.at[pl.ds(s * _NPS, _NPS)])
        pltpu.sync_copy(dst_hbm.at[pl.ds(wid * _NCH, _NCH)], idx_v)
        if counts_mode:
            pltpu.sync_copy(ones_hbm, rows_v)
        plsc.subcore_barrier()
        base = wid * _EW

        def loop(j, carry):
            if not counts_mode:
                pltpu.sync_copy(m_hbm.at[pl.ds(base + j * _C, _C)], rows_v)
            pltpu.sync_copy(rows_v, acc.at[idx_v.at[j]], add=True)
            return carry

        lax.fori_loop(0, _num_chunks(wid), loop, 0)
        plsc.subcore_barrier()
        pltpu.sync_copy(acc.at[pl.ds(s * _NPS, _NPS)],
                        out_hbm.at[c, pl.ds(s * _NPS, _NPS)])

    k = functools.partial(
        pl.kernel,
        mesh=_sc_mesh(),
        out_type=jax.ShapeDtypeStruct((_NC, _N, _D), jnp.float32),
        scratch_types=[
            pltpu.VMEM((_NCH, _C), jnp.int32),
            pltpu.VMEM((_C, _D), jnp.float32),
            pltpu.VMEM_SHARED((_N, _D), jnp.float32),
        ],
        compiler_params=_SC_PARAMS,
    )(body)
    return k(*operands)


def kernel(h_node, edge_index, h_edge, W1, b1, g1, be1, W2, b2, g2, be2,
           W_ih, W_hh, b_ih, b_hh):
    f32 = jnp.float32
    h_node = h_node.astype(f32)
    h_edge = h_edge.astype(f32)
    pad_e = _EP - _E
    src2 = jnp.concatenate(
        [edge_index[0], jnp.zeros((pad_e,), jnp.int32)]).reshape(
            _NW * _NCH, _C)
    dst2 = jnp.concatenate(
        [edge_index[1], jnp.zeros((pad_e,), jnp.int32)]).reshape(
            _NW * _NCH, _C)

    ecount = jnp.float32(_E)

    gram_h, colsum_h = _moments(h_edge, _DE)
    mh = (colsum_h[0] / ecount)
    sh = gram_h / ecount
    hp_ = jax.lax.Precision.HIGHEST
    w1mh = jnp.einsum('ck,k->c', W1, mh, precision=hp_)
    mu1 = w1mh + b1
    ey1sq = jnp.einsum('ck,kl,cl->c', W1, sh, W1, precision=hp_) \
        + 2.0 * b1 * w1mh + b1 * b1
    var1 = ey1sq - mu1 * mu1
    s1v = g1 / jnp.sqrt(var1 + _EPS)
    w1f = W1.T * s1v[None, :]
    b1f = ((b1 - mu1) * s1v + be1)[None, :]

    x1, gram_x, colsum_x = _x1_compute(h_edge, w1f, b1f)
    mx = colsum_x[0] / ecount
    sx = gram_x / ecount
    w2mx = jnp.einsum('ck,k->c', W2, mx, precision=hp_)
    mu2 = w2mx + b2
    ey2sq = jnp.einsum('ck,kl,cl->c', W2, sx, W2, precision=hp_) \
        + 2.0 * b2 * w2mx + b2 * b2
    var2 = ey2sq - mu2 * mu2
    alpha = g2 / jnp.sqrt(var2 + _EPS)
    cc = alpha * b2 + be2 - alpha * mu2
    a3 = (alpha[:, None] * W2).reshape(_D, _D, _DE)
    bmat = jnp.transpose(a3, (2, 0, 1)).reshape(_DE * _D, _D)
    cmat = cc.reshape(_D, _D)
    rx = jnp.kron(jnp.eye(_DE, dtype=f32), jnp.ones((1, _D), f32))
    rh = jnp.tile(jnp.eye(_D, dtype=f32), (1, _DE))

    zeros_nd = jnp.zeros((_N, _D), f32)
    ones_rows = jnp.ones((_C, _D), f32)

    cpart = _scatter_rows(None, dst2, zeros_nd, ones_rows)
    c0, c1 = cpart[0], cpart[1]

    wiht = W_ih.T.astype(f32)
    whht = W_hh.T.astype(f32)
    bih2 = b_ih[None, :].astype(f32)
    bhh2 = b_hh[None, :].astype(f32)

    h = h_node
    for _ in range(_N_LAYERS):
        hsrc = _gather_rows(h, src2)
        m = _edge_messages(hsrc, x1, rx, rh, bmat, cmat)
        spart = _scatter_rows(m, dst2, zeros_nd, ones_rows)
        h = _gru(spart[0], spart[1], c0, c1, h, wiht, whht, bih2, bhh2)
    return h


# speedup vs baseline: 3.8515x; 1.0248x over previous
import functools

import jax
import jax.numpy as jnp
from jax import lax
from jax.experimental import pallas as pl
from jax.experimental.pallas import tpu as pltpu
from jax.experimental.pallas import tpu_sc as plsc

_N = 10000
_E = 160000
_D = 32
_DE = 16
_N_LAYERS = 2
_EPS = 1e-5
_SLOPE = 0.8

_NC = 2
_NS = 16
_NW = _NC * _NS
_C = 128
_NCH = 40
_EW = _NCH * _C
_EP = _NW * _EW
_NCH_LAST = (_E - (_NW - 1) * _EW) // _C
_NPS = _N // _NS

_TE = 2000
_TEP = 4096
_TN = 2000


def _moments_body(x_ref, gram_ref, colsum_ref):
    @pl.when(pl.program_id(0) == 0)
    def _():
        gram_ref[...] = jnp.zeros_like(gram_ref)
        colsum_ref[...] = jnp.zeros_like(colsum_ref)

    x = x_ref[...]
    gram_ref[...] += lax.dot_general(
        x, x, (((0,), (0,)), ((), ())), preferred_element_type=jnp.float32)
    colsum_ref[...] += jnp.sum(x, axis=0, keepdims=True)


def _moments(x, d):
    grid = (x.shape[0] // _TE,)
    return pl.pallas_call(
        _moments_body,
        grid=grid,
        in_specs=[pl.BlockSpec((_TE, d), lambda i: (i, 0))],
        out_specs=[pl.BlockSpec((d, d), lambda i: (0, 0)),
                   pl.BlockSpec((1, d), lambda i: (0, 0))],
        out_shape=[jax.ShapeDtypeStruct((d, d), jnp.float32),
                   jax.ShapeDtypeStruct((1, d), jnp.float32)],
    )(x)


def _x1_body(he_ref, w_ref, b_ref, x1_ref, gram_ref, colsum_ref):
    @pl.when(pl.program_id(0) == 0)
    def _():
        gram_ref[...] = jnp.zeros_like(gram_ref)
        colsum_ref[...] = jnp.zeros_like(colsum_ref)

    y = jnp.dot(he_ref[...], w_ref[...],
                preferred_element_type=jnp.float32) + b_ref[...]
    x1 = jnp.where(y >= 0, y, _SLOPE * y)
    x1_ref[...] = x1
    gram_ref[...] += lax.dot_general(
        x1, x1, (((0,), (0,)), ((), ())), preferred_element_type=jnp.float32)
    colsum_ref[...] += jnp.sum(x1, axis=0, keepdims=True)


def _x1_compute(h_edge, w1f, b1f):
    grid = (_E // _TE,)
    return pl.pallas_call(
        _x1_body,
        grid=grid,
        in_specs=[pl.BlockSpec((_TE, _DE), lambda i: (i, 0)),
                  pl.BlockSpec((_DE, _DE), lambda i: (0, 0)),
                  pl.BlockSpec((1, _DE), lambda i: (0, 0))],
        out_specs=[pl.BlockSpec((_TE, _DE), lambda i: (i, 0)),
                   pl.BlockSpec((_DE, _DE), lambda i: (0, 0)),
                   pl.BlockSpec((1, _DE), lambda i: (0, 0))],
        out_shape=[jax.ShapeDtypeStruct((_EP, _DE), jnp.float32),
                   jax.ShapeDtypeStruct((_DE, _DE), jnp.float32),
                   jax.ShapeDtypeStruct((1, _DE), jnp.float32)],
    )(h_edge, w1f, b1f)


def _edge_body(hs_ref, x1_ref, rx_ref, b_ref, cm_ref, m_ref):
    hs = hs_ref[...]
    x1 = x1_ref[...]
    xr = jnp.dot(x1, rx_ref[...],
                 preferred_element_type=jnp.float32)
    hr = pltpu.repeat(hs, _DE, axis=1)
    p = xr * hr
    m_ref[...] = (
        jnp.dot(p, b_ref[...], preferred_element_type=jnp.float32)
        + jnp.dot(hs, cm_ref[...], preferred_element_type=jnp.float32))


def _edge_messages(hsrc, x1, rx, bmat, cmat):
    grid = (_EP // _TEP,)
    return pl.pallas_call(
        _edge_body,
        grid=grid,
        in_specs=[pl.BlockSpec((_TEP, _D), lambda i: (i, 0)),
                  pl.BlockSpec((_TEP, _DE), lambda i: (i, 0)),
                  pl.BlockSpec((_DE, _DE * _D), lambda i: (0, 0)),
                  pl.BlockSpec((_DE * _D, _D), lambda i: (0, 0)),
                  pl.BlockSpec((_D, _D), lambda i: (0, 0))],
        out_specs=pl.BlockSpec((_TEP, _D), lambda i: (i, 0)),
        out_shape=jax.ShapeDtypeStruct((_EP, _D), jnp.float32),
    )(hsrc, x1, rx, bmat, cmat)


def _gru_body(s0_ref, s1_ref, c0_ref, c1_ref, h_ref,
              wih_ref, whh_ref, bih_ref, bhh_ref, out_ref):
    cnt = jnp.maximum(c0_ref[...] + c1_ref[...], 1.0)
    mag = (s0_ref[...] + s1_ref[...]) / cnt
    h = h_ref[...]
    gi = jnp.dot(mag, wih_ref[...],
                 preferred_element_type=jnp.float32) + bih_ref[...]
    gh = jnp.dot(h, whh_ref[...],
                 preferred_element_type=jnp.float32) + bhh_ref[...]
    r = jax.nn.sigmoid(gi[:, 0:_D] + gh[:, 0:_D])
    zg = jax.nn.sigmoid(gi[:, _D:2 * _D] + gh[:, _D:2 * _D])
    n = jnp.tanh(gi[:, 2 * _D:3 * _D] + r * gh[:, 2 * _D:3 * _D])
    out_ref[...] = (1.0 - zg) * n + zg * h


def _gru(s0, s1, c0, c1, h, wiht, whht, bih2, bhh2):
    grid = (_N // _TN,)
    blk = lambda i: (i, 0)
    full = lambda i: (0, 0)
    return pl.pallas_call(
        _gru_body,
        grid=grid,
        in_specs=[pl.BlockSpec((_TN, _D), blk),
                  pl.BlockSpec((_TN, _D), blk),
                  pl.BlockSpec((_TN, _D), blk),
                  pl.BlockSpec((_TN, _D), blk),
                  pl.BlockSpec((_TN, _D), blk),
                  pl.BlockSpec((_D, 3 * _D), full),
                  pl.BlockSpec((_D, 3 * _D), full),
                  pl.BlockSpec((1, 3 * _D), full),
                  pl.BlockSpec((1, 3 * _D), full)],
        out_specs=pl.BlockSpec((_TN, _D), blk),
        out_shape=jax.ShapeDtypeStruct((_N, _D), jnp.float32),
    )(s0, s1, c0, c1, h, wiht, whht, bih2, bhh2)


def _sc_mesh():
    return plsc.VectorSubcoreMesh(core_axis_name="c", subcore_axis_name="s")


_SC_PARAMS = pltpu.CompilerParams(use_tc_tiling_on_sc=False)


def _num_chunks(wid):
    return jnp.where(wid == _NW - 1, _NCH_LAST, _NCH)


def _gather_rows(h, ei):
    @functools.partial(
        pl.kernel,
        mesh=_sc_mesh(),
        out_type=jax.ShapeDtypeStruct((_EP, _D), jnp.float32),
        scratch_types=[
            pltpu.VMEM((_NCH, _C), jnp.int32),
            pltpu.VMEM((_C, _D), jnp.float32),
            pltpu.SemaphoreType.DMA,
        ],
        compiler_params=_SC_PARAMS,
    )
    def k(h_hbm, ei_hbm, out_hbm, idx_v, rows_v, sem):
        c = lax.axis_index("c")
        s = lax.axis_index("s")
        wid = s * _NC + c
        base = wid * _EW

        def body(j, carry):
            pltpu.sync_copy(ei_hbm.at[0, pl.ds(base + j * _C, _C)],
                            idx_v.at[j])
            pltpu.async_copy(h_hbm.at[idx_v.at[j]], rows_v, sem).wait()
            pltpu.sync_copy(rows_v, out_hbm.at[pl.ds(base + j * _C, _C)])
            return carry

        lax.fori_loop(0, _num_chunks(wid), body, 0)

    return k(h, ei)


def _scatter_rows(rows, ei, zeros_nd, ones_rows):
    counts_mode = rows is None
    operands = (ei, zeros_nd, ones_rows) if counts_mode else (
        rows, ei, zeros_nd)

    def body(*refs):
        if counts_mode:
            ei_hbm, zero_hbm, ones_hbm, out_hbm, idx_v, rows_v, acc = refs
        else:
            m_hbm, ei_hbm, zero_hbm, out_hbm, idx_v, rows_v, acc = refs
        c = lax.axis_index("c")
        s = lax.axis_index("s")
        wid = s * _NC + c
        pltpu.sync_copy(zero_hbm.at[pl.ds(s * _NPS, _NPS)],
                        acc.at[pl.ds(s * _NPS, _NPS)])
        if counts_mode:
            pltpu.sync_copy(ones_hbm, rows_v)
        plsc.subcore_barrier()
        base = wid * _EW

        def loop(j, carry):
            pltpu.sync_copy(ei_hbm.at[1, pl.ds(base + j * _C, _C)],
                            idx_v.at[j])
            if not counts_mode:
                pltpu.sync_copy(m_hbm.at[pl.ds(base + j * _C, _C)], rows_v)
            pltpu.sync_copy(rows_v, acc.at[idx_v.at[j]], add=True)
            return carry

        lax.fori_loop(0, _num_chunks(wid), loop, 0)
        plsc.subcore_barrier()
        pltpu.sync_copy(acc.at[pl.ds(s * _NPS, _NPS)],
                        out_hbm.at[c, pl.ds(s * _NPS, _NPS)])

    k = functools.partial(
        pl.kernel,
        mesh=_sc_mesh(),
        out_type=jax.ShapeDtypeStruct((_NC, _N, _D), jnp.float32),
        scratch_types=[
            pltpu.VMEM((_NCH, _C), jnp.int32),
            pltpu.VMEM((_C, _D), jnp.float32),
            pltpu.VMEM_SHARED((_N, _D), jnp.float32),
        ],
        compiler_params=_SC_PARAMS,
    )(body)
    return k(*operands)


def kernel(h_node, edge_index, h_edge, W1, b1, g1, be1, W2, b2, g2, be2,
           W_ih, W_hh, b_ih, b_hh):
    f32 = jnp.float32
    h_node = h_node.astype(f32)
    h_edge = h_edge.astype(f32)
    ei = edge_index.astype(jnp.int32)

    ecount = jnp.float32(_E)

    gram_h, colsum_h = _moments(h_edge, _DE)
    mh = (colsum_h[0] / ecount)
    sh = gram_h / ecount
    hp_ = jax.lax.Precision.HIGHEST
    w1mh = jnp.einsum('ck,k->c', W1, mh, precision=hp_)
    mu1 = w1mh + b1
    ey1sq = jnp.einsum('ck,kl,cl->c', W1, sh, W1, precision=hp_) \
        + 2.0 * b1 * w1mh + b1 * b1
    var1 = ey1sq - mu1 * mu1
    s1v = g1 / jnp.sqrt(var1 + _EPS)
    w1f = W1.T * s1v[None, :]
    b1f = ((b1 - mu1) * s1v + be1)[None, :]

    x1, gram_x, colsum_x = _x1_compute(h_edge, w1f, b1f)
    mx = colsum_x[0] / ecount
    sx = gram_x / ecount
    w2mx = jnp.einsum('ck,k->c', W2, mx, precision=hp_)
    mu2 = w2mx + b2
    ey2sq = jnp.einsum('ck,kl,cl->c', W2, sx, W2, precision=hp_) \
        + 2.0 * b2 * w2mx + b2 * b2
    var2 = ey2sq - mu2 * mu2
    alpha = g2 / jnp.sqrt(var2 + _EPS)
    cc = alpha * b2 + be2 - alpha * mu2
    a3 = (alpha[:, None] * W2).reshape(_D, _D, _DE)
    bmat = jnp.transpose(a3, (2, 0, 1)).reshape(_DE * _D, _D)
    cmat = cc.reshape(_D, _D)
    rx = jnp.kron(jnp.eye(_DE, dtype=f32), jnp.ones((1, _D), f32))

    zeros_nd = jnp.zeros((_N, _D), f32)
    ones_rows = jnp.ones((_C, _D), f32)

    cpart = _scatter_rows(None, ei, zeros_nd, ones_rows)
    c0, c1 = cpart[0], cpart[1]

    wiht = W_ih.T.astype(f32)
    whht = W_hh.T.astype(f32)
    bih2 = b_ih[None, :].astype(f32)
    bhh2 = b_hh[None, :].astype(f32)

    h = h_node
    for _ in range(_N_LAYERS):
        hsrc = _gather_rows(h, ei)
        m = _edge_messages(hsrc, x1, rx, bmat, cmat)
        spart = _scatter_rows(m, ei, zeros_nd, ones_rows)
        h = _gru(spart[0], spart[1], c0, c1, h, wiht, whht, bih2, bhh2)
    return h


# speedup vs baseline: 4.2268x; 1.0974x over previous
import functools

import jax
import jax.numpy as jnp
from jax import lax
from jax.experimental import pallas as pl
from jax.experimental.pallas import tpu as pltpu
from jax.experimental.pallas import tpu_sc as plsc

_N = 10000
_E = 160000
_D = 32
_DE = 16
_N_LAYERS = 2
_EPS = 1e-5
_SLOPE = 0.8

_NC = 2
_NS = 16
_NW = _NC * _NS
_C = 128
_NCH = 40
_EW = _NCH * _C
_EP = _NW * _EW
_NCH_LAST = (_E - (_NW - 1) * _EW) // _C
_NPS = _N // _NS

_TE = 8000
_TEP = 4096
_TN = 2000


def _moments_body(x_ref, gram_ref, colsum_ref):
    @pl.when(pl.program_id(0) == 0)
    def _():
        gram_ref[...] = jnp.zeros_like(gram_ref)
        colsum_ref[...] = jnp.zeros_like(colsum_ref)

    x = x_ref[...]
    gram_ref[...] += lax.dot_general(
        x, x, (((0,), (0,)), ((), ())), preferred_element_type=jnp.float32)
    colsum_ref[...] += jnp.sum(x, axis=0, keepdims=True)


def _moments(x, d):
    grid = (x.shape[0] // _TE,)
    return pl.pallas_call(
        _moments_body,
        grid=grid,
        in_specs=[pl.BlockSpec((_TE, d), lambda i: (i, 0))],
        out_specs=[pl.BlockSpec((d, d), lambda i: (0, 0)),
                   pl.BlockSpec((1, d), lambda i: (0, 0))],
        out_shape=[jax.ShapeDtypeStruct((d, d), jnp.float32),
                   jax.ShapeDtypeStruct((1, d), jnp.float32)],
    )(x)


def _x1_body(he_ref, w_ref, b_ref, x1_ref, gram_ref, colsum_ref):
    @pl.when(pl.program_id(0) == 0)
    def _():
        gram_ref[...] = jnp.zeros_like(gram_ref)
        colsum_ref[...] = jnp.zeros_like(colsum_ref)

    y = jnp.dot(he_ref[...], w_ref[...],
                preferred_element_type=jnp.float32) + b_ref[...]
    x1 = jnp.where(y >= 0, y, _SLOPE * y)
    x1_ref[...] = x1.astype(jnp.bfloat16)
    gram_ref[...] += lax.dot_general(
        x1, x1, (((0,), (0,)), ((), ())), preferred_element_type=jnp.float32)
    colsum_ref[...] += jnp.sum(x1, axis=0, keepdims=True)


def _x1_compute(h_edge, w1f, b1f):
    grid = (_E // _TE,)
    return pl.pallas_call(
        _x1_body,
        grid=grid,
        in_specs=[pl.BlockSpec((_TE, _DE), lambda i: (i, 0)),
                  pl.BlockSpec((_DE, _DE), lambda i: (0, 0)),
                  pl.BlockSpec((1, _DE), lambda i: (0, 0))],
        out_specs=[pl.BlockSpec((_TE, _DE), lambda i: (i, 0)),
                   pl.BlockSpec((_DE, _DE), lambda i: (0, 0)),
                   pl.BlockSpec((1, _DE), lambda i: (0, 0))],
        out_shape=[jax.ShapeDtypeStruct((_EP, _DE), jnp.bfloat16),
                   jax.ShapeDtypeStruct((_DE, _DE), jnp.float32),
                   jax.ShapeDtypeStruct((1, _DE), jnp.float32)],
    )(h_edge, w1f, b1f)


def _edge_body(hs_ref, x1_ref, rx_ref, b_ref, cm_ref, m_ref):
    hs = hs_ref[...]
    x1 = x1_ref[...]
    xr = jnp.dot(x1, rx_ref[...],
                 preferred_element_type=jnp.float32)
    hr = pltpu.repeat(hs, _DE, axis=1)
    p = xr * hr
    m_ref[...] = (
        jnp.dot(p, b_ref[...], preferred_element_type=jnp.float32)
        + jnp.dot(hs, cm_ref[...], preferred_element_type=jnp.float32))


def _edge_messages(hsrc, x1, rx, bmat, cmat):
    grid = (_EP // _TEP,)
    return pl.pallas_call(
        _edge_body,
        grid=grid,
        in_specs=[pl.BlockSpec((_TEP, _D), lambda i: (i, 0)),
                  pl.BlockSpec((_TEP, _DE), lambda i: (i, 0)),
                  pl.BlockSpec((_DE, _DE * _D), lambda i: (0, 0)),
                  pl.BlockSpec((_DE * _D, _D), lambda i: (0, 0)),
                  pl.BlockSpec((_D, _D), lambda i: (0, 0))],
        out_specs=pl.BlockSpec((_TEP, _D), lambda i: (i, 0)),
        out_shape=jax.ShapeDtypeStruct((_EP, _D), jnp.float32),
    )(hsrc, x1, rx, bmat, cmat)


def _gru_body(s0_ref, s1_ref, c0_ref, c1_ref, h_ref,
              wih_ref, whh_ref, bih_ref, bhh_ref, out_ref):
    cnt = jnp.maximum(c0_ref[...] + c1_ref[...], 1.0)
    mag = (s0_ref[...] + s1_ref[...]) / cnt
    h = h_ref[...]
    gi = jnp.dot(mag, wih_ref[...],
                 preferred_element_type=jnp.float32) + bih_ref[...]
    gh = jnp.dot(h, whh_ref[...],
                 preferred_element_type=jnp.float32) + bhh_ref[...]
    r = jax.nn.sigmoid(gi[:, 0:_D] + gh[:, 0:_D])
    zg = jax.nn.sigmoid(gi[:, _D:2 * _D] + gh[:, _D:2 * _D])
    n = jnp.tanh(gi[:, 2 * _D:3 * _D] + r * gh[:, 2 * _D:3 * _D])
    out_ref[...] = (1.0 - zg) * n + zg * h


def _gru(s0, s1, c0, c1, h, wiht, whht, bih2, bhh2):
    grid = (_N // _TN,)
    blk = lambda i: (i, 0)
    full = lambda i: (0, 0)
    return pl.pallas_call(
        _gru_body,
        grid=grid,
        in_specs=[pl.BlockSpec((_TN, _D), blk),
                  pl.BlockSpec((_TN, _D), blk),
                  pl.BlockSpec((_TN, _D), blk),
                  pl.BlockSpec((_TN, _D), blk),
                  pl.BlockSpec((_TN, _D), blk),
                  pl.BlockSpec((_D, 3 * _D), full),
                  pl.BlockSpec((_D, 3 * _D), full),
                  pl.BlockSpec((1, 3 * _D), full),
                  pl.BlockSpec((1, 3 * _D), full)],
        out_specs=pl.BlockSpec((_TN, _D), blk),
        out_shape=jax.ShapeDtypeStruct((_N, _D), jnp.float32),
    )(s0, s1, c0, c1, h, wiht, whht, bih2, bhh2)


def _sc_mesh():
    return plsc.VectorSubcoreMesh(core_axis_name="c", subcore_axis_name="s")


_SC_PARAMS = pltpu.CompilerParams(use_tc_tiling_on_sc=False)


def _num_chunks(wid):
    return jnp.where(wid == _NW - 1, _NCH_LAST, _NCH)


def _gather_rows(h, ei):
    @functools.partial(
        pl.kernel,
        mesh=_sc_mesh(),
        out_type=jax.ShapeDtypeStruct((_EP, _D), jnp.float32),
        scratch_types=[
            pltpu.VMEM((_NCH, _C), jnp.int32),
            pltpu.VMEM((_C, _D), jnp.float32),
            pltpu.SemaphoreType.DMA,
        ],
        compiler_params=_SC_PARAMS,
    )
    def k(h_hbm, ei_hbm, out_hbm, idx_v, rows_v, sem):
        c = lax.axis_index("c")
        s = lax.axis_index("s")
        wid = s * _NC + c
        base = wid * _EW

        def body(j, carry):
            pltpu.sync_copy(ei_hbm.at[0, pl.ds(base + j * _C, _C)],
                            idx_v.at[j])
            pltpu.async_copy(h_hbm.at[idx_v.at[j]], rows_v, sem).wait()
            pltpu.sync_copy(rows_v, out_hbm.at[pl.ds(base + j * _C, _C)])
            return carry

        lax.fori_loop(0, _num_chunks(wid), body, 0)

    return k(h, ei)


def _scatter_rows(rows, ei, zeros_nd, ones_rows):
    counts_mode = rows is None
    operands = (ei, zeros_nd, ones_rows) if counts_mode else (
        rows, ei, zeros_nd)

    def body(*refs):
        if counts_mode:
            ei_hbm, zero_hbm, ones_hbm, out_hbm, idx_v, rows_v, acc = refs
        else:
            m_hbm, ei_hbm, zero_hbm, out_hbm, idx_v, rows_v, acc = refs
        c = lax.axis_index("c")
        s = lax.axis_index("s")
        wid = s * _NC + c
        pltpu.sync_copy(zero_hbm.at[pl.ds(s * _NPS, _NPS)],
                        acc.at[pl.ds(s * _NPS, _NPS)])
        if counts_mode:
            pltpu.sync_copy(ones_hbm, rows_v)
        plsc.subcore_barrier()
        base = wid * _EW

        def loop(j, carry):
            pltpu.sync_copy(ei_hbm.at[1, pl.ds(base + j * _C, _C)],
                            idx_v.at[j])
            if not counts_mode:
                pltpu.sync_copy(m_hbm.at[pl.ds(base + j * _C, _C)], rows_v)
            pltpu.sync_copy(rows_v, acc.at[idx_v.at[j]], add=True)
            return carry

        lax.fori_loop(0, _num_chunks(wid), loop, 0)
        plsc.subcore_barrier()
        pltpu.sync_copy(acc.at[pl.ds(s * _NPS, _NPS)],
                        out_hbm.at[c, pl.ds(s * _NPS, _NPS)])

    k = functools.partial(
        pl.kernel,
        mesh=_sc_mesh(),
        out_type=jax.ShapeDtypeStruct((_NC, _N, _D), jnp.float32),
        scratch_types=[
            pltpu.VMEM((_NCH, _C), jnp.int32),
            pltpu.VMEM((_C, _D), jnp.float32),
            pltpu.VMEM_SHARED((_N, _D), jnp.float32),
        ],
        compiler_params=_SC_PARAMS,
    )(body)
    return k(*operands)


def kernel(h_node, edge_index, h_edge, W1, b1, g1, be1, W2, b2, g2, be2,
           W_ih, W_hh, b_ih, b_hh):
    f32 = jnp.float32
    h_node = h_node.astype(f32)
    h_edge = h_edge.astype(f32)
    ei = edge_index.astype(jnp.int32)

    ecount = jnp.float32(_E)

    gram_h, colsum_h = _moments(h_edge, _DE)
    mh = (colsum_h[0] / ecount)
    sh = gram_h / ecount
    hp_ = jax.lax.Precision.HIGHEST
    w1mh = jnp.einsum('ck,k->c', W1, mh, precision=hp_)
    mu1 = w1mh + b1
    ey1sq = jnp.einsum('ck,kl,cl->c', W1, sh, W1, precision=hp_) \
        + 2.0 * b1 * w1mh + b1 * b1
    var1 = ey1sq - mu1 * mu1
    s1v = g1 / jnp.sqrt(var1 + _EPS)
    w1f = W1.T * s1v[None, :]
    b1f = ((b1 - mu1) * s1v + be1)[None, :]

    x1, gram_x, colsum_x = _x1_compute(h_edge, w1f, b1f)
    mx = colsum_x[0] / ecount
    sx = gram_x / ecount
    w2mx = jnp.einsum('ck,k->c', W2, mx, precision=hp_)
    mu2 = w2mx + b2
    ey2sq = jnp.einsum('ck,kl,cl->c', W2, sx, W2, precision=hp_) \
        + 2.0 * b2 * w2mx + b2 * b2
    var2 = ey2sq - mu2 * mu2
    alpha = g2 / jnp.sqrt(var2 + _EPS)
    cc = alpha * b2 + be2 - alpha * mu2
    a3 = (alpha[:, None] * W2).reshape(_D, _D, _DE)
    bmat = jnp.transpose(a3, (2, 0, 1)).reshape(_DE * _D, _D)
    cmat = cc.reshape(_D, _D)
    rx = jnp.kron(jnp.eye(_DE), jnp.ones((1, _D))).astype(jnp.bfloat16)

    zeros_nd = jnp.zeros((_N, _D), f32)
    ones_rows = jnp.ones((_C, _D), f32)

    cpart = _scatter_rows(None, ei, zeros_nd, ones_rows)
    c0, c1 = cpart[0], cpart[1]

    wiht = W_ih.T.astype(f32)
    whht = W_hh.T.astype(f32)
    bih2 = b_ih[None, :].astype(f32)
    bhh2 = b_hh[None, :].astype(f32)

    h = h_node
    for _ in range(_N_LAYERS):
        hsrc = _gather_rows(h, ei)
        m = _edge_messages(hsrc, x1, rx, bmat, cmat)
        spart = _scatter_rows(m, ei, zeros_nd, ones_rows)
        h = _gru(spart[0], spart[1], c0, c1, h, wiht, whht, bih2, bhh2)
    return h


# speedup vs baseline: 5.7764x; 1.3666x over previous
import functools

import jax
import jax.numpy as jnp
from jax import lax
from jax.experimental import pallas as pl
from jax.experimental.pallas import tpu as pltpu
from jax.experimental.pallas import tpu_sc as plsc

_N = 10000
_E = 160000
_D = 32
_DE = 16
_N_LAYERS = 2
_EPS = 1e-5
_SLOPE = 0.8

_NC = 2
_NS = 16
_NW = _NC * _NS
_C = 128
_NCH = 40
_EW = _NCH * _C
_EP = _NW * _EW
_NCH_LAST = (_E - (_NW - 1) * _EW) // _C
_NPS = _N // _NS

_TE = 8000
_TEP = 4096
_TN = 2000


def _moments_body(x_ref, gram_ref, colsum_ref):
    @pl.when(pl.program_id(0) == 0)
    def _():
        gram_ref[...] = jnp.zeros_like(gram_ref)
        colsum_ref[...] = jnp.zeros_like(colsum_ref)

    x = x_ref[...]
    gram_ref[...] += lax.dot_general(
        x, x, (((0,), (0,)), ((), ())), preferred_element_type=jnp.float32)
    colsum_ref[...] += jnp.sum(x, axis=0, keepdims=True)


def _moments(x, d):
    grid = (x.shape[0] // _TE,)
    return pl.pallas_call(
        _moments_body,
        grid=grid,
        in_specs=[pl.BlockSpec((_TE, d), lambda i: (i, 0))],
        out_specs=[pl.BlockSpec((d, d), lambda i: (0, 0)),
                   pl.BlockSpec((1, d), lambda i: (0, 0))],
        out_shape=[jax.ShapeDtypeStruct((d, d), jnp.float32),
                   jax.ShapeDtypeStruct((1, d), jnp.float32)],
    )(x)


def _x1_body(he_ref, w_ref, b_ref, x1_ref, gram_ref, colsum_ref):
    @pl.when(pl.program_id(0) == 0)
    def _():
        gram_ref[...] = jnp.zeros_like(gram_ref)
        colsum_ref[...] = jnp.zeros_like(colsum_ref)

    y = jnp.dot(he_ref[...], w_ref[...],
                preferred_element_type=jnp.float32) + b_ref[...]
    x1 = jnp.where(y >= 0, y, _SLOPE * y)
    x1_ref[...] = x1.astype(jnp.bfloat16)
    gram_ref[...] += lax.dot_general(
        x1, x1, (((0,), (0,)), ((), ())), preferred_element_type=jnp.float32)
    colsum_ref[...] += jnp.sum(x1, axis=0, keepdims=True)


def _x1_compute(h_edge, w1f, b1f):
    grid = (_E // _TE,)
    return pl.pallas_call(
        _x1_body,
        grid=grid,
        in_specs=[pl.BlockSpec((_TE, _DE), lambda i: (i, 0)),
                  pl.BlockSpec((_DE, _DE), lambda i: (0, 0)),
                  pl.BlockSpec((1, _DE), lambda i: (0, 0))],
        out_specs=[pl.BlockSpec((_TE, _DE), lambda i: (i, 0)),
                   pl.BlockSpec((_DE, _DE), lambda i: (0, 0)),
                   pl.BlockSpec((1, _DE), lambda i: (0, 0))],
        out_shape=[jax.ShapeDtypeStruct((_EP, _DE), jnp.bfloat16),
                   jax.ShapeDtypeStruct((_DE, _DE), jnp.float32),
                   jax.ShapeDtypeStruct((1, _DE), jnp.float32)],
    )(h_edge, w1f, b1f)


def _edge_body(hs_ref, x1_ref, rx_ref, b_ref, cm_ref, m_ref):
    hs = hs_ref[:, 0:_D]
    x1 = x1_ref[...]
    xr = jnp.dot(x1, rx_ref[...],
                 preferred_element_type=jnp.float32)
    hr = pltpu.repeat(hs, _DE, axis=1)
    p = xr * hr
    m_ref[:, 0:_D] = (
        jnp.dot(p, b_ref[...], preferred_element_type=jnp.float32)
        + jnp.dot(hs, cm_ref[...], preferred_element_type=jnp.float32))


def _edge_messages(hsrc, x1, rx, bmat, cmat):
    grid = (_EP // _TEP,)
    return pl.pallas_call(
        _edge_body,
        grid=grid,
        in_specs=[pl.BlockSpec((_TEP, 128), lambda i: (i, 0)),
                  pl.BlockSpec((_TEP, _DE), lambda i: (i, 0)),
                  pl.BlockSpec((_DE, _DE * _D), lambda i: (0, 0)),
                  pl.BlockSpec((_DE * _D, _D), lambda i: (0, 0)),
                  pl.BlockSpec((_D, _D), lambda i: (0, 0))],
        out_specs=pl.BlockSpec((_TEP, 128), lambda i: (i, 0)),
        out_shape=jax.ShapeDtypeStruct((_EP, 128), jnp.float32),
    )(hsrc, x1, rx, bmat, cmat)


def _gru_body(s0_ref, s1_ref, c0_ref, c1_ref, h_ref,
              wih_ref, whh_ref, bih_ref, bhh_ref, out_ref):
    cnt = jnp.maximum(c0_ref[:, 0:_D] + c1_ref[:, 0:_D], 1.0)
    mag = (s0_ref[:, 0:_D] + s1_ref[:, 0:_D]) / cnt
    h = h_ref[...]
    gi = jnp.dot(mag, wih_ref[...],
                 preferred_element_type=jnp.float32) + bih_ref[...]
    gh = jnp.dot(h, whh_ref[...],
                 preferred_element_type=jnp.float32) + bhh_ref[...]
    r = jax.nn.sigmoid(gi[:, 0:_D] + gh[:, 0:_D])
    zg = jax.nn.sigmoid(gi[:, _D:2 * _D] + gh[:, _D:2 * _D])
    n = jnp.tanh(gi[:, 2 * _D:3 * _D] + r * gh[:, 2 * _D:3 * _D])
    out_ref[...] = (1.0 - zg) * n + zg * h


def _gru(s0, s1, c0, c1, h, wiht, whht, bih2, bhh2):
    grid = (_N // _TN,)
    blk = lambda i: (i, 0)
    full = lambda i: (0, 0)
    return pl.pallas_call(
        _gru_body,
        grid=grid,
        in_specs=[pl.BlockSpec((_TN, 128), blk),
                  pl.BlockSpec((_TN, 128), blk),
                  pl.BlockSpec((_TN, 128), blk),
                  pl.BlockSpec((_TN, 128), blk),
                  pl.BlockSpec((_TN, _D), blk),
                  pl.BlockSpec((_D, 3 * _D), full),
                  pl.BlockSpec((_D, 3 * _D), full),
                  pl.BlockSpec((1, 3 * _D), full),
                  pl.BlockSpec((1, 3 * _D), full)],
        out_specs=pl.BlockSpec((_TN, _D), blk),
        out_shape=jax.ShapeDtypeStruct((_N, _D), jnp.float32),
    )(s0, s1, c0, c1, h, wiht, whht, bih2, bhh2)


def _sc_mesh():
    return plsc.VectorSubcoreMesh(core_axis_name="c", subcore_axis_name="s")


_SC_PARAMS = pltpu.CompilerParams(use_tc_tiling_on_sc=False)


def _num_chunks(wid):
    return jnp.where(wid == _NW - 1, _NCH_LAST, _NCH)


def _gather_rows(h, ei):
    @functools.partial(
        pl.kernel,
        mesh=_sc_mesh(),
        out_type=jax.ShapeDtypeStruct((_EP, 128), jnp.float32),
        scratch_types=[
            pltpu.VMEM((_NCH, _C), jnp.int32),
            pltpu.VMEM((_C, _D), jnp.float32),
            pltpu.SemaphoreType.DMA,
        ],
        compiler_params=_SC_PARAMS,
    )
    def k(h_hbm, ei_hbm, out_hbm, idx_v, rows_v, sem):
        c = lax.axis_index("c")
        s = lax.axis_index("s")
        wid = s * _NC + c
        base = wid * _EW

        def body(j, carry):
            pltpu.sync_copy(ei_hbm.at[0, pl.ds(base + j * _C, _C)],
                            idx_v.at[j])
            pltpu.async_copy(h_hbm.at[idx_v.at[j]], rows_v, sem).wait()
            pltpu.sync_copy(rows_v,
                            out_hbm.at[pl.ds(base + j * _C, _C),
                                       pl.ds(0, _D)])
            return carry

        lax.fori_loop(0, _num_chunks(wid), body, 0)

    return k(h, ei)


def _scatter_rows(rows, ei, zeros_nd, ones_rows):
    counts_mode = rows is None
    operands = (ei, zeros_nd, ones_rows) if counts_mode else (
        rows, ei, zeros_nd)

    def body(*refs):
        if counts_mode:
            ei_hbm, zero_hbm, ones_hbm, out_hbm, idx_v, rows_v, acc = refs
        else:
            m_hbm, ei_hbm, zero_hbm, out_hbm, idx_v, rows_v, acc = refs
        c = lax.axis_index("c")
        s = lax.axis_index("s")
        wid = s * _NC + c
        pltpu.sync_copy(zero_hbm.at[pl.ds(s * _NPS, _NPS)],
                        acc.at[pl.ds(s * _NPS, _NPS)])
        if counts_mode:
            pltpu.sync_copy(ones_hbm, rows_v)
        plsc.subcore_barrier()
        base = wid * _EW

        def loop(j, carry):
            pltpu.sync_copy(ei_hbm.at[1, pl.ds(base + j * _C, _C)],
                            idx_v.at[j])
            if not counts_mode:
                pltpu.sync_copy(m_hbm.at[pl.ds(base + j * _C, _C),
                                         pl.ds(0, _D)], rows_v)
            pltpu.sync_copy(rows_v, acc.at[idx_v.at[j]], add=True)
            return carry

        lax.fori_loop(0, _num_chunks(wid), loop, 0)
        plsc.subcore_barrier()
        pltpu.sync_copy(acc.at[pl.ds(s * _NPS, _NPS)],
                        out_hbm.at[c, pl.ds(s * _NPS, _NPS), pl.ds(0, _D)])

    k = functools.partial(
        pl.kernel,
        mesh=_sc_mesh(),
        out_type=jax.ShapeDtypeStruct((_NC, _N, 128), jnp.float32),
        scratch_types=[
            pltpu.VMEM((_NCH, _C), jnp.int32),
            pltpu.VMEM((_C, _D), jnp.float32),
            pltpu.VMEM_SHARED((_N, _D), jnp.float32),
        ],
        compiler_params=_SC_PARAMS,
    )(body)
    return k(*operands)


def kernel(h_node, edge_index, h_edge, W1, b1, g1, be1, W2, b2, g2, be2,
           W_ih, W_hh, b_ih, b_hh):
    f32 = jnp.float32
    h_node = h_node.astype(f32)
    h_edge = h_edge.astype(f32)
    ei = edge_index.astype(jnp.int32)

    ecount = jnp.float32(_E)

    gram_h, colsum_h = _moments(h_edge, _DE)
    mh = (colsum_h[0] / ecount)
    sh = gram_h / ecount
    hp_ = jax.lax.Precision.HIGHEST
    w1mh = jnp.einsum('ck,k->c', W1, mh, precision=hp_)
    mu1 = w1mh + b1
    ey1sq = jnp.einsum('ck,kl,cl->c', W1, sh, W1, precision=hp_) \
        + 2.0 * b1 * w1mh + b1 * b1
    var1 = ey1sq - mu1 * mu1
    s1v = g1 / jnp.sqrt(var1 + _EPS)
    w1f = W1.T * s1v[None, :]
    b1f = ((b1 - mu1) * s1v + be1)[None, :]

    x1, gram_x, colsum_x = _x1_compute(h_edge, w1f, b1f)
    mx = colsum_x[0] / ecount
    sx = gram_x / ecount
    w2mx = jnp.einsum('ck,k->c', W2, mx, precision=hp_)
    mu2 = w2mx + b2
    ey2sq = jnp.einsum('ck,kl,cl->c', W2, sx, W2, precision=hp_) \
        + 2.0 * b2 * w2mx + b2 * b2
    var2 = ey2sq - mu2 * mu2
    alpha = g2 / jnp.sqrt(var2 + _EPS)
    cc = alpha * b2 + be2 - alpha * mu2
    a3 = (alpha[:, None] * W2).reshape(_D, _D, _DE)
    bmat = jnp.transpose(a3, (2, 0, 1)).reshape(_DE * _D, _D)
    cmat = cc.reshape(_D, _D)
    rx = jnp.kron(jnp.eye(_DE), jnp.ones((1, _D))).astype(jnp.bfloat16)

    zeros_nd = jnp.zeros((_N, _D), f32)
    ones_rows = jnp.ones((_C, _D), f32)

    cpart = _scatter_rows(None, ei, zeros_nd, ones_rows)
    c0, c1 = cpart[0], cpart[1]

    wiht = W_ih.T.astype(f32)
    whht = W_hh.T.astype(f32)
    bih2 = b_ih[None, :].astype(f32)
    bhh2 = b_hh[None, :].astype(f32)

    h = h_node
    for _ in range(_N_LAYERS):
        hsrc = _gather_rows(h, ei)
        m = _edge_messages(hsrc, x1, rx, bmat, cmat)
        spart = _scatter_rows(m, ei, zeros_nd, ones_rows)
        h = _gru(spart[0], spart[1], c0, c1, h, wiht, whht, bih2, bhh2)
    return h


# speedup vs baseline: 5.9383x; 1.0280x over previous
import functools

import jax
import jax.numpy as jnp
from jax import lax
from jax.experimental import pallas as pl
from jax.experimental.pallas import tpu as pltpu
from jax.experimental.pallas import tpu_sc as plsc

_N = 10000
_E = 160000
_D = 32
_DE = 16
_N_LAYERS = 2
_EPS = 1e-5
_SLOPE = 0.8

_NC = 2
_NS = 16
_NW = _NC * _NS
_C = 128
_NCH = 40
_EW = _NCH * _C
_EP = _NW * _EW
_NCH_LAST = (_E - (_NW - 1) * _EW) // _C
_NPS = _N // _NS

_TE = 8000
_TEP = 4096
_TN = 2000


def _moments_body(x_ref, gram_ref, colsum_ref):
    @pl.when(pl.program_id(0) == 0)
    def _():
        gram_ref[...] = jnp.zeros_like(gram_ref)
        colsum_ref[...] = jnp.zeros_like(colsum_ref)

    x = x_ref[...]
    gram_ref[...] += lax.dot_general(
        x, x, (((0,), (0,)), ((), ())), preferred_element_type=jnp.float32)
    colsum_ref[...] += jnp.sum(x, axis=0, keepdims=True)


def _moments(x, d):
    grid = (x.shape[0] // _TE,)
    return pl.pallas_call(
        _moments_body,
        grid=grid,
        in_specs=[pl.BlockSpec((_TE, d), lambda i: (i, 0))],
        out_specs=[pl.BlockSpec((d, d), lambda i: (0, 0)),
                   pl.BlockSpec((1, d), lambda i: (0, 0))],
        out_shape=[jax.ShapeDtypeStruct((d, d), jnp.float32),
                   jax.ShapeDtypeStruct((1, d), jnp.float32)],
    )(x)


def _x1_body(he_ref, w_ref, b_ref, x1_ref, gram_ref, colsum_ref):
    @pl.when(pl.program_id(0) == 0)
    def _():
        gram_ref[...] = jnp.zeros_like(gram_ref)
        colsum_ref[...] = jnp.zeros_like(colsum_ref)

    y = jnp.dot(he_ref[...], w_ref[...],
                preferred_element_type=jnp.float32) + b_ref[...]
    x1 = jnp.where(y >= 0, y, _SLOPE * y)
    x1_ref[...] = x1.astype(jnp.bfloat16)
    gram_ref[...] += lax.dot_general(
        x1, x1, (((0,), (0,)), ((), ())), preferred_element_type=jnp.float32)
    colsum_ref[...] += jnp.sum(x1, axis=0, keepdims=True)


def _x1_compute(h_edge, w1f, b1f):
    grid = (_E // _TE,)
    return pl.pallas_call(
        _x1_body,
        grid=grid,
        in_specs=[pl.BlockSpec((_TE, _DE), lambda i: (i, 0)),
                  pl.BlockSpec((_DE, _DE), lambda i: (0, 0)),
                  pl.BlockSpec((1, _DE), lambda i: (0, 0))],
        out_specs=[pl.BlockSpec((_TE, _DE), lambda i: (i, 0)),
                   pl.BlockSpec((_DE, _DE), lambda i: (0, 0)),
                   pl.BlockSpec((1, _DE), lambda i: (0, 0))],
        out_shape=[jax.ShapeDtypeStruct((_EP, _DE), jnp.bfloat16),
                   jax.ShapeDtypeStruct((_DE, _DE), jnp.float32),
                   jax.ShapeDtypeStruct((1, _DE), jnp.float32)],
    )(h_edge, w1f, b1f)


def _edge_body(hs_ref, x1_ref, rx_ref, b_ref, cm_ref, m_ref):
    hs = hs_ref[:, 0:_D]
    x1 = x1_ref[...]
    xr = jnp.dot(x1, rx_ref[...],
                 preferred_element_type=jnp.float32)
    hr = pltpu.repeat(hs, _DE, axis=1)
    p = xr * hr
    m_ref[:, 0:_D] = (
        jnp.dot(p, b_ref[...], preferred_element_type=jnp.float32)
        + jnp.dot(hs, cm_ref[...], preferred_element_type=jnp.float32))


def _edge_messages(hsrc, x1, rx, bmat, cmat, half):
    hb = _EW // 2
    blk = lambda i, h=half: (2 * i + h, 0)
    return pl.pallas_call(
        _edge_body,
        grid=(_NW,),
        in_specs=[pl.BlockSpec((hb, 128), blk),
                  pl.BlockSpec((hb, _DE), blk),
                  pl.BlockSpec((_DE, _DE * _D), lambda i: (0, 0)),
                  pl.BlockSpec((_DE * _D, _D), lambda i: (0, 0)),
                  pl.BlockSpec((_D, _D), lambda i: (0, 0))],
        out_specs=pl.BlockSpec((hb, 128), blk),
        out_shape=jax.ShapeDtypeStruct((_EP, 128), jnp.float32),
    )(hsrc, x1, rx, bmat, cmat)


def _gru_body(sa0_ref, sa1_ref, sb0_ref, sb1_ref, c0_ref, c1_ref, h_ref,
              wih_ref, whh_ref, bih_ref, bhh_ref, out_ref):
    cnt = jnp.maximum(c0_ref[:, 0:_D] + c1_ref[:, 0:_D], 1.0)
    mag = ((sa0_ref[:, 0:_D] + sa1_ref[:, 0:_D])
           + (sb0_ref[:, 0:_D] + sb1_ref[:, 0:_D])) / cnt
    h = h_ref[...]
    gi = jnp.dot(mag, wih_ref[...],
                 preferred_element_type=jnp.float32) + bih_ref[...]
    gh = jnp.dot(h, whh_ref[...],
                 preferred_element_type=jnp.float32) + bhh_ref[...]
    r = jax.nn.sigmoid(gi[:, 0:_D] + gh[:, 0:_D])
    zg = jax.nn.sigmoid(gi[:, _D:2 * _D] + gh[:, _D:2 * _D])
    n = jnp.tanh(gi[:, 2 * _D:3 * _D] + r * gh[:, 2 * _D:3 * _D])
    out_ref[...] = (1.0 - zg) * n + zg * h


def _gru(sa0, sa1, sb0, sb1, c0, c1, h, wiht, whht, bih2, bhh2):
    grid = (_N // _TN,)
    blk = lambda i: (i, 0)
    full = lambda i: (0, 0)
    return pl.pallas_call(
        _gru_body,
        grid=grid,
        in_specs=[pl.BlockSpec((_TN, 128), blk),
                  pl.BlockSpec((_TN, 128), blk),
                  pl.BlockSpec((_TN, 128), blk),
                  pl.BlockSpec((_TN, 128), blk),
                  pl.BlockSpec((_TN, 128), blk),
                  pl.BlockSpec((_TN, 128), blk),
                  pl.BlockSpec((_TN, _D), blk),
                  pl.BlockSpec((_D, 3 * _D), full),
                  pl.BlockSpec((_D, 3 * _D), full),
                  pl.BlockSpec((1, 3 * _D), full),
                  pl.BlockSpec((1, 3 * _D), full)],
        out_specs=pl.BlockSpec((_TN, _D), blk),
        out_shape=jax.ShapeDtypeStruct((_N, _D), jnp.float32),
    )(sa0, sa1, sb0, sb1, c0, c1, h, wiht, whht, bih2, bhh2)


def _sc_mesh():
    return plsc.VectorSubcoreMesh(core_axis_name="c", subcore_axis_name="s")


_SC_PARAMS = pltpu.CompilerParams(use_tc_tiling_on_sc=False)


def _num_chunks(wid):
    return jnp.where(wid == _NW - 1, _NCH_LAST, _NCH)


_HC = _NCH // 2


def _half_bounds(wid, half):
    lo = half * _HC
    hi = jnp.minimum(_num_chunks(wid), lo + _HC)
    return lo, jnp.maximum(hi, lo)


def _gather_rows(h, ei, half):
    @functools.partial(
        pl.kernel,
        mesh=_sc_mesh(),
        out_type=jax.ShapeDtypeStruct((_EP, 128), jnp.float32),
        scratch_types=[
            pltpu.VMEM((_NCH, _C), jnp.int32),
            pltpu.VMEM((_C, _D), jnp.float32),
            pltpu.SemaphoreType.DMA,
        ],
        compiler_params=_SC_PARAMS,
    )
    def k(h_hbm, ei_hbm, out_hbm, idx_v, rows_v, sem):
        c = lax.axis_index("c")
        s = lax.axis_index("s")
        wid = s * _NC + c
        base = wid * _EW

        def body(j, carry):
            pltpu.sync_copy(ei_hbm.at[0, pl.ds(base + j * _C, _C)],
                            idx_v.at[j])
            pltpu.async_copy(h_hbm.at[idx_v.at[j]], rows_v, sem).wait()
            pltpu.sync_copy(rows_v,
                            out_hbm.at[pl.ds(base + j * _C, _C),
                                       pl.ds(0, _D)])
            return carry

        lo, hi = _half_bounds(wid, half)
        lax.fori_loop(lo, hi, body, 0)

    return k(h, ei)


def _scatter_rows(rows, ei, zeros_nd, ones_rows, half=None):
    counts_mode = rows is None
    operands = (ei, zeros_nd, ones_rows) if counts_mode else (
        rows, ei, zeros_nd)

    def body(*refs):
        if counts_mode:
            ei_hbm, zero_hbm, ones_hbm, out_hbm, idx_v, rows_v, acc = refs
        else:
            m_hbm, ei_hbm, zero_hbm, out_hbm, idx_v, rows_v, acc = refs
        c = lax.axis_index("c")
        s = lax.axis_index("s")
        wid = s * _NC + c
        pltpu.sync_copy(zero_hbm.at[pl.ds(s * _NPS, _NPS)],
                        acc.at[pl.ds(s * _NPS, _NPS)])
        if counts_mode:
            pltpu.sync_copy(ones_hbm, rows_v)
        plsc.subcore_barrier()
        base = wid * _EW

        def loop(j, carry):
            pltpu.sync_copy(ei_hbm.at[1, pl.ds(base + j * _C, _C)],
                            idx_v.at[j])
            if not counts_mode:
                pltpu.sync_copy(m_hbm.at[pl.ds(base + j * _C, _C),
                                         pl.ds(0, _D)], rows_v)
            pltpu.sync_copy(rows_v, acc.at[idx_v.at[j]], add=True)
            return carry

        if half is None:
            lax.fori_loop(0, _num_chunks(wid), loop, 0)
        else:
            lo, hi = _half_bounds(wid, half)
            lax.fori_loop(lo, hi, loop, 0)
        plsc.subcore_barrier()
        pltpu.sync_copy(acc.at[pl.ds(s * _NPS, _NPS)],
                        out_hbm.at[c, pl.ds(s * _NPS, _NPS), pl.ds(0, _D)])

    k = functools.partial(
        pl.kernel,
        mesh=_sc_mesh(),
        out_type=jax.ShapeDtypeStruct((_NC, _N, 128), jnp.float32),
        scratch_types=[
            pltpu.VMEM((_NCH, _C), jnp.int32),
            pltpu.VMEM((_C, _D), jnp.float32),
            pltpu.VMEM_SHARED((_N, _D), jnp.float32),
        ],
        compiler_params=_SC_PARAMS,
    )(body)
    return k(*operands)


def kernel(h_node, edge_index, h_edge, W1, b1, g1, be1, W2, b2, g2, be2,
           W_ih, W_hh, b_ih, b_hh):
    f32 = jnp.float32
    h_node = h_node.astype(f32)
    h_edge = h_edge.astype(f32)
    ei = edge_index.astype(jnp.int32)

    ecount = jnp.float32(_E)

    gram_h, colsum_h = _moments(h_edge, _DE)
    mh = (colsum_h[0] / ecount)
    sh = gram_h / ecount
    hp_ = jax.lax.Precision.HIGHEST
    w1mh = jnp.einsum('ck,k->c', W1, mh, precision=hp_)
    mu1 = w1mh + b1
    ey1sq = jnp.einsum('ck,kl,cl->c', W1, sh, W1, precision=hp_) \
        + 2.0 * b1 * w1mh + b1 * b1
    var1 = ey1sq - mu1 * mu1
    s1v = g1 / jnp.sqrt(var1 + _EPS)
    w1f = W1.T * s1v[None, :]
    b1f = ((b1 - mu1) * s1v + be1)[None, :]

    x1, gram_x, colsum_x = _x1_compute(h_edge, w1f, b1f)
    mx = colsum_x[0] / ecount
    sx = gram_x / ecount
    w2mx = jnp.einsum('ck,k->c', W2, mx, precision=hp_)
    mu2 = w2mx + b2
    ey2sq = jnp.einsum('ck,kl,cl->c', W2, sx, W2, precision=hp_) \
        + 2.0 * b2 * w2mx + b2 * b2
    var2 = ey2sq - mu2 * mu2
    alpha = g2 / jnp.sqrt(var2 + _EPS)
    cc = alpha * b2 + be2 - alpha * mu2
    a3 = (alpha[:, None] * W2).reshape(_D, _D, _DE)
    bmat = jnp.transpose(a3, (2, 0, 1)).reshape(_DE * _D, _D)
    cmat = cc.reshape(_D, _D)
    rx = jnp.kron(jnp.eye(_DE), jnp.ones((1, _D))).astype(jnp.bfloat16)

    zeros_nd = jnp.zeros((_N, _D), f32)
    ones_rows = jnp.ones((_C, _D), f32)

    cpart = _scatter_rows(None, ei, zeros_nd, ones_rows)
    c0, c1 = cpart[0], cpart[1]

    wiht = W_ih.T.astype(f32)
    whht = W_hh.T.astype(f32)
    bih2 = b_ih[None, :].astype(f32)
    bhh2 = b_hh[None, :].astype(f32)

    h = h_node
    for _ in range(_N_LAYERS):
        hsrc_a = _gather_rows(h, ei, 0)
        hsrc_b = _gather_rows(h, ei, 1)
        m_a = _edge_messages(hsrc_a, x1, rx, bmat, cmat, 0)
        m_b = _edge_messages(hsrc_b, x1, rx, bmat, cmat, 1)
        sp_a = _scatter_rows(m_a, ei, zeros_nd, ones_rows, 0)
        sp_b = _scatter_rows(m_b, ei, zeros_nd, ones_rows, 1)
        h = _gru(sp_a[0], sp_a[1], sp_b[0], sp_b[1], c0, c1, h,
                 wiht, whht, bih2, bhh2)
    return h


# speedup vs baseline: 6.1038x; 1.0279x over previous
import functools

import jax
import jax.numpy as jnp
from jax import lax
from jax.experimental import pallas as pl
from jax.experimental.pallas import tpu as pltpu
from jax.experimental.pallas import tpu_sc as plsc

_N = 10000
_E = 160000
_D = 32
_DE = 16
_N_LAYERS = 2
_EPS = 1e-5
_SLOPE = 0.8

_NC = 2
_NS = 16
_NW = _NC * _NS
_C = 128
_NCH = 40
_EW = _NCH * _C
_EP = _NW * _EW
_NCH_LAST = (_E - (_NW - 1) * _EW) // _C
_NPS = _N // _NS

_TE = 8000
_TEP = 4096
_TN = 2000


def _moments_body(x_ref, gram_ref, colsum_ref):
    @pl.when(pl.program_id(0) == 0)
    def _():
        gram_ref[...] = jnp.zeros_like(gram_ref)
        colsum_ref[...] = jnp.zeros_like(colsum_ref)

    x = x_ref[...]
    gram_ref[...] += lax.dot_general(
        x, x, (((0,), (0,)), ((), ())), preferred_element_type=jnp.float32)
    colsum_ref[...] += jnp.sum(x, axis=0, keepdims=True)


def _moments(x, d):
    grid = (x.shape[0] // _TE,)
    return pl.pallas_call(
        _moments_body,
        grid=grid,
        in_specs=[pl.BlockSpec((_TE, d), lambda i: (i, 0))],
        out_specs=[pl.BlockSpec((d, d), lambda i: (0, 0)),
                   pl.BlockSpec((1, d), lambda i: (0, 0))],
        out_shape=[jax.ShapeDtypeStruct((d, d), jnp.float32),
                   jax.ShapeDtypeStruct((1, d), jnp.float32)],
    )(x)


def _x1_body(he_ref, w_ref, b_ref, x1_ref, gram_ref, colsum_ref):
    @pl.when(pl.program_id(0) == 0)
    def _():
        gram_ref[...] = jnp.zeros_like(gram_ref)
        colsum_ref[...] = jnp.zeros_like(colsum_ref)

    y = jnp.dot(he_ref[...], w_ref[...],
                preferred_element_type=jnp.float32) + b_ref[...]
    x1 = jnp.where(y >= 0, y, _SLOPE * y)
    x1_ref[...] = x1.astype(jnp.bfloat16)
    gram_ref[...] += lax.dot_general(
        x1, x1, (((0,), (0,)), ((), ())), preferred_element_type=jnp.float32)
    colsum_ref[...] += jnp.sum(x1, axis=0, keepdims=True)


def _x1_compute(h_edge, w1f, b1f):
    grid = (_E // _TE,)
    return pl.pallas_call(
        _x1_body,
        grid=grid,
        in_specs=[pl.BlockSpec((_TE, _DE), lambda i: (i, 0)),
                  pl.BlockSpec((_DE, _DE), lambda i: (0, 0)),
                  pl.BlockSpec((1, _DE), lambda i: (0, 0))],
        out_specs=[pl.BlockSpec((_TE, _DE), lambda i: (i, 0)),
                   pl.BlockSpec((_DE, _DE), lambda i: (0, 0)),
                   pl.BlockSpec((1, _DE), lambda i: (0, 0))],
        out_shape=[jax.ShapeDtypeStruct((_EP, _DE), jnp.bfloat16),
                   jax.ShapeDtypeStruct((_DE, _DE), jnp.float32),
                   jax.ShapeDtypeStruct((1, _DE), jnp.float32)],
    )(h_edge, w1f, b1f)


def _edge_body(hs_ref, x1_ref, rx_ref, b_ref, cm_ref, m_ref):
    hs = hs_ref[:, 0:_D]
    x1 = x1_ref[...]
    xr = jnp.dot(x1, rx_ref[...],
                 preferred_element_type=jnp.float32)
    hr = pltpu.repeat(hs, _DE, axis=1)
    p = xr * hr
    m_ref[:, 0:_D] = (
        jnp.dot(p, b_ref[...], preferred_element_type=jnp.float32)
        + jnp.dot(hs, cm_ref[...], preferred_element_type=jnp.float32))


def _edge_messages(hsrc, x1, rx, bmat, cmat, half):
    hb = _EW // 2
    blk = lambda i, h=half: (2 * i + h, 0)
    return pl.pallas_call(
        _edge_body,
        grid=(_NW,),
        in_specs=[pl.BlockSpec((hb, 128), blk),
                  pl.BlockSpec((hb, _DE), blk),
                  pl.BlockSpec((_DE, _DE * _D), lambda i: (0, 0)),
                  pl.BlockSpec((_DE * _D, _D), lambda i: (0, 0)),
                  pl.BlockSpec((_D, _D), lambda i: (0, 0))],
        out_specs=pl.BlockSpec((hb, 128), blk),
        out_shape=jax.ShapeDtypeStruct((_EP, 128), jnp.float32),
    )(hsrc, x1, rx, bmat, cmat)


def _gru_core(mag, h, wih_ref, whh_ref, bih_ref, bhh_ref):
    gi = jnp.dot(mag, wih_ref[...],
                 preferred_element_type=jnp.float32) + bih_ref[...]
    gh = jnp.dot(h, whh_ref[...],
                 preferred_element_type=jnp.float32) + bhh_ref[...]
    r = jax.nn.sigmoid(gi[:, 0:_D] + gh[:, 0:_D])
    zg = jax.nn.sigmoid(gi[:, _D:2 * _D] + gh[:, _D:2 * _D])
    n = jnp.tanh(gi[:, 2 * _D:3 * _D] + r * gh[:, 2 * _D:3 * _D])
    return (1.0 - zg) * n + zg * h


def _gru1_body(sa0_ref, sa1_ref, sb0_ref, sb1_ref, h_ref,
               wih_ref, whh_ref, bih_ref, bhh_ref, out_ref, cnt_ref):
    ssum = ((sa0_ref[:, 0:_D] + sa1_ref[:, 0:_D])
            + (sb0_ref[:, 0:_D] + sb1_ref[:, 0:_D]))
    cnt = jnp.maximum((sa0_ref[:, _D:2 * _D] + sa1_ref[:, _D:2 * _D])
                      + (sb0_ref[:, _D:2 * _D] + sb1_ref[:, _D:2 * _D]),
                      1.0)
    cnt_ref[:, 0:_D] = cnt
    out_ref[...] = _gru_core(ssum / cnt, h_ref[...],
                             wih_ref, whh_ref, bih_ref, bhh_ref)


def _gru2_body(sa0_ref, sa1_ref, sb0_ref, sb1_ref, cnt_ref, h_ref,
               wih_ref, whh_ref, bih_ref, bhh_ref, out_ref):
    ssum = ((sa0_ref[:, 0:_D] + sa1_ref[:, 0:_D])
            + (sb0_ref[:, 0:_D] + sb1_ref[:, 0:_D]))
    out_ref[...] = _gru_core(ssum / cnt_ref[:, 0:_D], h_ref[...],
                             wih_ref, whh_ref, bih_ref, bhh_ref)


def _gru(sa0, sa1, sb0, sb1, cnt, h, wiht, whht, bih2, bhh2):
    grid = (_N // _TN,)
    blk = lambda i: (i, 0)
    full = lambda i: (0, 0)
    wide = [pl.BlockSpec((_TN, 128), blk)] * 4
    tail = [pl.BlockSpec((_TN, _D), blk),
            pl.BlockSpec((_D, 3 * _D), full),
            pl.BlockSpec((_D, 3 * _D), full),
            pl.BlockSpec((1, 3 * _D), full),
            pl.BlockSpec((1, 3 * _D), full)]
    if cnt is None:
        return pl.pallas_call(
            _gru1_body,
            grid=grid,
            in_specs=wide + tail,
            out_specs=[pl.BlockSpec((_TN, _D), blk),
                       pl.BlockSpec((_TN, 128), blk)],
            out_shape=[jax.ShapeDtypeStruct((_N, _D), jnp.float32),
                       jax.ShapeDtypeStruct((_N, 128), jnp.float32)],
        )(sa0, sa1, sb0, sb1, h, wiht, whht, bih2, bhh2)
    return pl.pallas_call(
        _gru2_body,
        grid=grid,
        in_specs=wide + [pl.BlockSpec((_TN, 128), blk)] + tail,
        out_specs=pl.BlockSpec((_TN, _D), blk),
        out_shape=jax.ShapeDtypeStruct((_N, _D), jnp.float32),
    )(sa0, sa1, sb0, sb1, cnt, h, wiht, whht, bih2, bhh2)


def _sc_mesh():
    return plsc.VectorSubcoreMesh(core_axis_name="c", subcore_axis_name="s")


_SC_PARAMS = pltpu.CompilerParams(use_tc_tiling_on_sc=False)


def _num_chunks(wid):
    return jnp.where(wid == _NW - 1, _NCH_LAST, _NCH)


_HC = _NCH // 2


def _half_bounds(wid, half):
    lo = half * _HC
    hi = jnp.minimum(_num_chunks(wid), lo + _HC)
    return lo, jnp.maximum(hi, lo)


def _gather_rows(h, ei, half):
    @functools.partial(
        pl.kernel,
        mesh=_sc_mesh(),
        out_type=jax.ShapeDtypeStruct((_EP, 128), jnp.float32),
        scratch_types=[
            pltpu.VMEM((_NCH, _C), jnp.int32),
            pltpu.VMEM((_C, _D), jnp.float32),
            pltpu.SemaphoreType.DMA,
        ],
        compiler_params=_SC_PARAMS,
    )
    def k(h_hbm, ei_hbm, out_hbm, idx_v, rows_v, sem):
        c = lax.axis_index("c")
        s = lax.axis_index("s")
        wid = s * _NC + c
        base = wid * _EW

        def body(j, carry):
            pltpu.sync_copy(ei_hbm.at[0, pl.ds(base + j * _C, _C)],
                            idx_v.at[j])
            pltpu.async_copy(h_hbm.at[idx_v.at[j]], rows_v, sem).wait()
            pltpu.sync_copy(rows_v,
                            out_hbm.at[pl.ds(base + j * _C, _C),
                                       pl.ds(0, _D)])
            return carry

        lo, hi = _half_bounds(wid, half)
        lax.fori_loop(lo, hi, body, 0)

    return k(h, ei)


def _scatter_rows(rows, ei, zeros_nd, ones_rows, half, with_counts):
    def body(m_hbm, ei_hbm, zero_hbm, ones_hbm, out_hbm,
             idx_v, rows_v, ones_v, acc, acc2):
        c = lax.axis_index("c")
        s = lax.axis_index("s")
        wid = s * _NC + c
        pltpu.sync_copy(zero_hbm.at[pl.ds(s * _NPS, _NPS)],
                        acc.at[pl.ds(s * _NPS, _NPS)])
        if with_counts:
            pltpu.sync_copy(zero_hbm.at[pl.ds(s * _NPS, _NPS)],
                            acc2.at[pl.ds(s * _NPS, _NPS)])
            pltpu.sync_copy(ones_hbm, ones_v)
        plsc.subcore_barrier()
        base = wid * _EW

        def loop(j, carry):
            pltpu.sync_copy(ei_hbm.at[1, pl.ds(base + j * _C, _C)],
                            idx_v.at[j])
            pltpu.sync_copy(m_hbm.at[pl.ds(base + j * _C, _C),
                                     pl.ds(0, _D)], rows_v)
            pltpu.sync_copy(rows_v, acc.at[idx_v.at[j]], add=True)
            if with_counts:
                pltpu.sync_copy(ones_v, acc2.at[idx_v.at[j]], add=True)
            return carry

        lo, hi = _half_bounds(wid, half)
        lax.fori_loop(lo, hi, loop, 0)
        plsc.subcore_barrier()
        pltpu.sync_copy(acc.at[pl.ds(s * _NPS, _NPS)],
                        out_hbm.at[c, pl.ds(s * _NPS, _NPS), pl.ds(0, _D)])
        if with_counts:
            pltpu.sync_copy(acc2.at[pl.ds(s * _NPS, _NPS)],
                            out_hbm.at[c, pl.ds(s * _NPS, _NPS),
                                       pl.ds(_D, _D)])

    k = functools.partial(
        pl.kernel,
        mesh=_sc_mesh(),
        out_type=jax.ShapeDtypeStruct((_NC, _N, 128), jnp.float32),
        scratch_types=[
            pltpu.VMEM((_NCH, _C), jnp.int32),
            pltpu.VMEM((_C, _D), jnp.float32),
            pltpu.VMEM((_C, _D), jnp.float32),
            pltpu.VMEM_SHARED((_N, _D), jnp.float32),
            pltpu.VMEM_SHARED((_N, _D), jnp.float32),
        ],
        compiler_params=_SC_PARAMS,
    )(body)
    return k(rows, ei, zeros_nd, ones_rows)


def kernel(h_node, edge_index, h_edge, W1, b1, g1, be1, W2, b2, g2, be2,
           W_ih, W_hh, b_ih, b_hh):
    f32 = jnp.float32
    h_node = h_node.astype(f32)
    h_edge = h_edge.astype(f32)
    ei = edge_index.astype(jnp.int32)

    ecount = jnp.float32(_E)

    gram_h, colsum_h = _moments(h_edge, _DE)
    mh = (colsum_h[0] / ecount)
    sh = gram_h / ecount
    hp_ = jax.lax.Precision.HIGHEST
    w1mh = jnp.einsum('ck,k->c', W1, mh, precision=hp_)
    mu1 = w1mh + b1
    ey1sq = jnp.einsum('ck,kl,cl->c', W1, sh, W1, precision=hp_) \
        + 2.0 * b1 * w1mh + b1 * b1
    var1 = ey1sq - mu1 * mu1
    s1v = g1 / jnp.sqrt(var1 + _EPS)
    w1f = W1.T * s1v[None, :]
    b1f = ((b1 - mu1) * s1v + be1)[None, :]

    x1, gram_x, colsum_x = _x1_compute(h_edge, w1f, b1f)
    mx = colsum_x[0] / ecount
    sx = gram_x / ecount
    w2mx = jnp.einsum('ck,k->c', W2, mx, precision=hp_)
    mu2 = w2mx + b2
    ey2sq = jnp.einsum('ck,kl,cl->c', W2, sx, W2, precision=hp_) \
        + 2.0 * b2 * w2mx + b2 * b2
    var2 = ey2sq - mu2 * mu2
    alpha = g2 / jnp.sqrt(var2 + _EPS)
    cc = alpha * b2 + be2 - alpha * mu2
    a3 = (alpha[:, None] * W2).reshape(_D, _D, _DE)
    bmat = jnp.transpose(a3, (2, 0, 1)).reshape(_DE * _D, _D)
    cmat = cc.reshape(_D, _D)
    rx = jnp.kron(jnp.eye(_DE), jnp.ones((1, _D))).astype(jnp.bfloat16)

    zeros_nd = jnp.zeros((_N, _D), f32)
    ones_rows = jnp.ones((_C, _D), f32)

    wiht = W_ih.T.astype(f32)
    whht = W_hh.T.astype(f32)
    bih2 = b_ih[None, :].astype(f32)
    bhh2 = b_hh[None, :].astype(f32)

    h = h_node
    cnt = None
    for layer in range(_N_LAYERS):
        hsrc_a = _gather_rows(h, ei, 0)
        hsrc_b = _gather_rows(h, ei, 1)
        m_a = _edge_messages(hsrc_a, x1, rx, bmat, cmat, 0)
        m_b = _edge_messages(hsrc_b, x1, rx, bmat, cmat, 1)
        wc = layer == 0
        sp_a = _scatter_rows(m_a, ei, zeros_nd, ones_rows, 0, wc)
        sp_b = _scatter_rows(m_b, ei, zeros_nd, ones_rows, 1, wc)
        out = _gru(sp_a[0], sp_a[1], sp_b[0], sp_b[1], cnt, h,
                   wiht, whht, bih2, bhh2)
        if wc:
            h, cnt = out
        else:
            h = out
    return h
